# baseline (speedup 1.0000x reference)
import functools

import jax
import jax.numpy as jnp
from jax import lax
from jax.experimental import pallas as pl
from jax.experimental.pallas import tpu as pltpu
from jax.experimental.pallas import tpu_sc as plsc

_N = 20000
_TILES = 16
_RPT = 1280
_GROUPS = _RPT // 16
_EPS = 1e-7

_mesh = plsc.VectorSubcoreMesh(core_axis_name="c", subcore_axis_name="s",
                               num_cores=1)


@functools.partial(
    pl.kernel,
    mesh=_mesh,
    compiler_params=pltpu.CompilerParams(
        needs_layout_passes=False,
        use_tc_tiling_on_sc=False,
        skip_device_barrier=True,
        disable_bounds_checks=True,
        disable_semaphore_checks=True,
    ),
    out_type=jax.ShapeDtypeStruct((_TILES, 16), jnp.float32),
    scratch_types=[
        pltpu.VMEM((_RPT, 4), jnp.float32),
        pltpu.VMEM((_RPT, 4), jnp.float32),
        pltpu.VMEM((16,), jnp.float32),
        pltpu.VMEM_SHARED((_TILES, 16), jnp.float32),
        pltpu.VMEM((_TILES, 16), jnp.float32),
        pltpu.VMEM((16,), jnp.float32),
    ],
)
def _giou_sc(pred_hbm, tgt_hbm, out_hbm, pred_v, tgt_v, acc_v, shared,
             red_v, res_v):
    sid = lax.axis_index("s")
    lo = sid * _RPT
    b = jnp.minimum(lo, _N - _RPT)
    pltpu.sync_copy(pred_hbm.at[pl.ds(b, _RPT), :], pred_v)
    pltpu.sync_copy(tgt_hbm.at[pl.ds(b, _RPT), :], tgt_v)

    lane = lax.iota(jnp.int32, 16)
    c0 = jnp.zeros((16,), jnp.int32)
    c1 = c0 + 1
    c2 = c0 + 2
    c3 = c0 + 3

    def body(g, acc):
        r = g * 16 + lane
        px1 = plsc.load_gather(pred_v, [r, c0])
        py1 = plsc.load_gather(pred_v, [r, c1])
        px2 = plsc.load_gather(pred_v, [r, c2])
        py2 = plsc.load_gather(pred_v, [r, c3])
        tx1 = plsc.load_gather(tgt_v, [r, c0])
        ty1 = plsc.load_gather(tgt_v, [r, c1])
        tx2 = plsc.load_gather(tgt_v, [r, c2])
        ty2 = plsc.load_gather(tgt_v, [r, c3])
        iw = jnp.maximum(jnp.minimum(px2, tx2) - jnp.maximum(px1, tx1), 0.0)
        ih = jnp.maximum(jnp.minimum(py2, ty2) - jnp.maximum(py1, ty1), 0.0)
        inter = iw * ih
        area_p = (px2 - px1) * (py2 - py1)
        area_t = (tx2 - tx1) * (ty2 - ty1)
        union = area_p + area_t - inter
        iou = inter / (union + _EPS)
        cw = jnp.maximum(px2, tx2) - jnp.minimum(px1, tx1)
        ch = jnp.maximum(py2, ty2) - jnp.minimum(py1, ty1)
        area_c = cw * ch
        giou = iou - (area_c - union) / (area_c + _EPS)
        loss = 1.0 - giou
        row = b + g * 16 + lane
        return acc + jnp.where(row >= lo, loss, 0.0)

    acc = lax.fori_loop(0, _GROUPS, body, jnp.zeros((16,), jnp.float32))

    acc_v[...] = acc
    pltpu.sync_copy(acc_v, out_hbm.at[sid])


def kernel(pred_boxes, target_boxes):
    out = _giou_sc(pred_boxes, target_boxes)
    return (jnp.sum(out) * (1.0 / _N))[None]


# speedup vs baseline: 2.5701x; 2.5701x over previous
import functools

import jax
import jax.numpy as jnp
from jax import lax
from jax.experimental import pallas as pl
from jax.experimental.pallas import tpu as pltpu
from jax.experimental.pallas import tpu_sc as plsc

_N = 20000
_TILES = 16
_RPT = 1280
_GROUPS = _RPT // 16
_EPS = 1e-7

_mesh = plsc.VectorSubcoreMesh(core_axis_name="c", subcore_axis_name="s",
                               num_cores=1)


@functools.partial(
    pl.kernel,
    mesh=_mesh,
    compiler_params=pltpu.CompilerParams(
        needs_layout_passes=False,
        use_tc_tiling_on_sc=False,
        skip_device_barrier=True,
        disable_bounds_checks=True,
        disable_semaphore_checks=True,
    ),
    out_type=jax.ShapeDtypeStruct((_TILES, 16), jnp.float32),
    scratch_types=[
        pltpu.VMEM((4, _RPT), jnp.float32),
        pltpu.VMEM((4, _RPT), jnp.float32),
        pltpu.VMEM((16,), jnp.float32),
    ],
)
def _giou_sc(pred_hbm, tgt_hbm, out_hbm, pred_v, tgt_v, acc_v):
    sid = lax.axis_index("s")
    lo = sid * _RPT
    b = jnp.minimum(lo, _N - _RPT)
    for f in range(4):
        pltpu.sync_copy(pred_hbm.at[pl.ds(f * _N + b, _RPT)], pred_v.at[f])
        pltpu.sync_copy(tgt_hbm.at[pl.ds(f * _N + b, _RPT)], tgt_v.at[f])

    lane = lax.iota(jnp.int32, 16)

    def body(g, acc):
        s = pl.ds(g * 16, 16)
        px1 = pred_v[0, s]
        py1 = pred_v[1, s]
        px2 = pred_v[2, s]
        py2 = pred_v[3, s]
        tx1 = tgt_v[0, s]
        ty1 = tgt_v[1, s]
        tx2 = tgt_v[2, s]
        ty2 = tgt_v[3, s]
        iw = jnp.maximum(jnp.minimum(px2, tx2) - jnp.maximum(px1, tx1), 0.0)
        ih = jnp.maximum(jnp.minimum(py2, ty2) - jnp.maximum(py1, ty1), 0.0)
        inter = iw * ih
        area_p = (px2 - px1) * (py2 - py1)
        area_t = (tx2 - tx1) * (ty2 - ty1)
        union = area_p + area_t - inter
        iou = inter / (union + _EPS)
        cw = jnp.maximum(px2, tx2) - jnp.minimum(px1, tx1)
        ch = jnp.maximum(py2, ty2) - jnp.minimum(py1, ty1)
        area_c = cw * ch
        giou = iou - (area_c - union) / (area_c + _EPS)
        loss = 1.0 - giou
        row = b + g * 16 + lane
        return acc + jnp.where(row >= lo, loss, 0.0)

    acc = lax.fori_loop(0, _GROUPS, body, jnp.zeros((16,), jnp.float32))

    acc_v[...] = acc
    pltpu.sync_copy(acc_v, out_hbm.at[sid])


def kernel(pred_boxes, target_boxes):
    out = _giou_sc(jnp.transpose(pred_boxes).ravel(),
                   jnp.transpose(target_boxes).ravel())
    return (jnp.sum(out) * (1.0 / _N))[None]


# speedup vs baseline: 3.2594x; 1.2682x over previous
import functools

import jax
import jax.numpy as jnp
from jax import lax
from jax.experimental import pallas as pl
from jax.experimental.pallas import tpu as pltpu
from jax.experimental.pallas import tpu_sc as plsc

_N = 20000
_TILES = 16
_RPT = 1280
_GROUPS = _RPT // 16
_EPS = 1e-7

_mesh = plsc.VectorSubcoreMesh(core_axis_name="c", subcore_axis_name="s",
                               num_cores=1)


@functools.partial(
    pl.kernel,
    mesh=_mesh,
    compiler_params=pltpu.CompilerParams(
        needs_layout_passes=False,
        use_tc_tiling_on_sc=False,
        skip_device_barrier=True,
        disable_bounds_checks=True,
        disable_semaphore_checks=True,
    ),
    out_type=(jax.ShapeDtypeStruct((_TILES, 16), jnp.float32),
              jax.ShapeDtypeStruct((16,), jnp.float32)),
    scratch_types=[
        pltpu.VMEM((4, _RPT), jnp.float32),
        pltpu.VMEM((4, _RPT), jnp.float32),
        pltpu.VMEM((16,), jnp.float32),
        pltpu.VMEM((_TILES, 16), jnp.float32),
        pltpu.VMEM((16,), jnp.float32),
        pltpu.SemaphoreType.DMA,
    ],
)
def _giou_sc(pred_hbm, tgt_hbm, part_hbm, out_hbm, pred_v, tgt_v, acc_v,
             red_v, res_v, sem):
    sid = lax.axis_index("s")
    lo = sid * _RPT
    b = jnp.minimum(lo, _N - _RPT)
    copies = []
    for f in range(4):
        copies.append(pltpu.async_copy(
            pred_hbm.at[pl.ds(f * _N + b, _RPT)], pred_v.at[f], sem))
        copies.append(pltpu.async_copy(
            tgt_hbm.at[pl.ds(f * _N + b, _RPT)], tgt_v.at[f], sem))
    for c in copies:
        c.wait()

    lane = lax.iota(jnp.int32, 16)

    def body(g, acc):
        s = pl.ds(g * 16, 16)
        px1 = pred_v[0, s]
        py1 = pred_v[1, s]
        px2 = pred_v[2, s]
        py2 = pred_v[3, s]
        tx1 = tgt_v[0, s]
        ty1 = tgt_v[1, s]
        tx2 = tgt_v[2, s]
        ty2 = tgt_v[3, s]
        iw = jnp.maximum(jnp.minimum(px2, tx2) - jnp.maximum(px1, tx1), 0.0)
        ih = jnp.maximum(jnp.minimum(py2, ty2) - jnp.maximum(py1, ty1), 0.0)
        inter = iw * ih
        area_p = (px2 - px1) * (py2 - py1)
        area_t = (tx2 - tx1) * (ty2 - ty1)
        union = area_p + area_t - inter
        iou = inter / (union + _EPS)
        cw = jnp.maximum(px2, tx2) - jnp.minimum(px1, tx1)
        ch = jnp.maximum(py2, ty2) - jnp.minimum(py1, ty1)
        area_c = cw * ch
        giou = iou - (area_c - union) / (area_c + _EPS)
        loss = 1.0 - giou
        row = b + g * 16 + lane
        return acc + jnp.where(row >= lo, loss, 0.0)

    acc = lax.fori_loop(0, _GROUPS, body, jnp.zeros((16,), jnp.float32))

    acc_v[...] = acc
    pltpu.sync_copy(acc_v, part_hbm.at[sid])
    plsc.subcore_barrier()

    @pl.when(sid == 0)
    def _():
        pltpu.sync_copy(part_hbm, red_v)
        tot = red_v[0]
        for j in range(1, _TILES):
            tot = tot + red_v[j]
        res_v[...] = jnp.broadcast_to(jnp.sum(tot) * (1.0 / _N), (16,))
        pltpu.sync_copy(res_v, out_hbm)


def kernel(pred_boxes, target_boxes):
    _, out = _giou_sc(jnp.transpose(pred_boxes).ravel(),
                      jnp.transpose(target_boxes).ravel())
    return out[:1]


# speedup vs baseline: 3.2611x; 1.0005x over previous
import functools

import jax
import jax.numpy as jnp
from jax import lax
from jax.experimental import pallas as pl
from jax.experimental.pallas import tpu as pltpu
from jax.experimental.pallas import tpu_sc as plsc

_N = 20000
_TILES = 16
_RPT = 1280
_GROUPS = _RPT // 16
_EPS = 1e-7

_mesh = plsc.VectorSubcoreMesh(core_axis_name="c", subcore_axis_name="s",
                               num_cores=1)


@functools.partial(
    pl.kernel,
    mesh=_mesh,
    compiler_params=pltpu.CompilerParams(
        needs_layout_passes=False,
        use_tc_tiling_on_sc=False,
        skip_device_barrier=True,
        disable_bounds_checks=True,
        disable_semaphore_checks=True,
    ),
    out_type=(jax.ShapeDtypeStruct((_TILES, 16), jnp.float32),
              jax.ShapeDtypeStruct((16,), jnp.float32)),
    scratch_types=[
        pltpu.VMEM((4, _RPT), jnp.float32),
        pltpu.VMEM((4, _RPT), jnp.float32),
        pltpu.VMEM((16,), jnp.float32),
        pltpu.VMEM((_TILES, 16), jnp.float32),
        pltpu.VMEM((16,), jnp.float32),
        pltpu.SemaphoreType.DMA,
    ],
)
def _giou_sc(pred_hbm, tgt_hbm, part_hbm, out_hbm, pred_v, tgt_v, acc_v,
             red_v, res_v, sem):
    sid = lax.axis_index("s")
    lo = sid * _RPT
    b = jnp.minimum(lo, _N - _RPT)
    copies = []
    for f in range(4):
        copies.append(pltpu.async_copy(
            pred_hbm.at[pl.ds(f * _N + b, _RPT)], pred_v.at[f], sem))
        copies.append(pltpu.async_copy(
            tgt_hbm.at[pl.ds(f * _N + b, _RPT)], tgt_v.at[f], sem))
    for c in copies:
        c.wait()

    lane = lax.iota(jnp.int32, 16)

    def body(g, acc):
        s = pl.ds(g * 16, 16)
        px1 = pred_v[0, s]
        py1 = pred_v[1, s]
        px2 = pred_v[2, s]
        py2 = pred_v[3, s]
        tx1 = tgt_v[0, s]
        ty1 = tgt_v[1, s]
        tx2 = tgt_v[2, s]
        ty2 = tgt_v[3, s]
        iw = jnp.maximum(jnp.minimum(px2, tx2) - jnp.maximum(px1, tx1), 0.0)
        ih = jnp.maximum(jnp.minimum(py2, ty2) - jnp.maximum(py1, ty1), 0.0)
        inter = iw * ih
        area_p = (px2 - px1) * (py2 - py1)
        area_t = (tx2 - tx1) * (ty2 - ty1)
        union = area_p + area_t - inter
        iou = inter / (union + _EPS)
        cw = jnp.maximum(px2, tx2) - jnp.minimum(px1, tx1)
        ch = jnp.maximum(py2, ty2) - jnp.minimum(py1, ty1)
        area_c = cw * ch
        giou = iou - (area_c - union) / (area_c + _EPS)
        loss = 1.0 - giou
        row = b + g * 16 + lane
        return acc + jnp.where(row >= lo, loss, 0.0)

    acc = lax.fori_loop(0, _GROUPS, body, jnp.zeros((16,), jnp.float32),
                        unroll=4)

    acc_v[...] = acc
    pltpu.sync_copy(acc_v, part_hbm.at[sid])
    plsc.subcore_barrier()

    @pl.when(sid == 0)
    def _():
        pltpu.sync_copy(part_hbm, red_v)
        tot = red_v[0]
        for j in range(1, _TILES):
            tot = tot + red_v[j]
        res_v[...] = jnp.broadcast_to(jnp.sum(tot) * (1.0 / _N), (16,))
        pltpu.sync_copy(res_v, out_hbm)


def kernel(pred_boxes, target_boxes):
    _, out = _giou_sc(jnp.transpose(pred_boxes).ravel(),
                      jnp.transpose(target_boxes).ravel())
    return out[:1]
